# baseline (speedup 1.0000x reference)
import jax
import jax.numpy as jnp
from jax import lax
from jax.experimental import pallas as pl
from jax.experimental.pallas import tpu as pltpu
from jax.experimental.pallas import tpu_sc as plsc

N_NODES = 10000
N_EDGES = 320000
IN_FEAT = 10000
OUT_FEAT = 128
NUM_RELS = 16

NC = 2
NS = 16
NW = NC * NS
EPW = N_EDGES // NW
K = 80
NCHUNK = EPW // K
NZCHUNK = N_NODES // K


def _sc_kernel(embed, idx2, norm2, dst3, part,
               idx_v, norm_v, dstc_v, rows_v, acc, gsem, dsem, ssem):
    cid = lax.axis_index("c")
    sid = lax.axis_index("s")
    wid = cid * NS + sid

    pltpu.sync_copy(idx2.at[wid], idx_v)
    pltpu.sync_copy(norm2.at[wid], norm_v)

    def zrow(r, carry):
        for j in range(OUT_FEAT // 16):
            rows_v[0, r, pl.ds(16 * j, 16)] = jnp.zeros((16,), jnp.float32)
        return carry
    lax.fori_loop(0, K, zrow, 0)
    nzc = (NZCHUNK - sid + NS - 1) // NS

    def zcopy(t, carry):
        j = sid + t * NS
        pltpu.sync_copy(rows_v.at[0], acc.at[pl.ds(j * K, K)])
        return carry
    lax.fori_loop(0, nzc, zcopy, 0)

    plsc.subcore_barrier()

    def g_desc(c, b):
        return pltpu.make_async_copy(
            embed.at[idx_v.at[pl.ds(c * K, K)]], rows_v.at[b], gsem.at[b])

    def d_desc(c, b):
        return pltpu.make_async_copy(dst3.at[wid, c], dstc_v.at[b],
                                     dsem.at[b])

    def s_desc(b):
        return pltpu.make_async_copy(rows_v.at[b], acc.at[dstc_v.at[b]],
                                     ssem.at[b])

    g_desc(0, 0).start()
    d_desc(0, 0).start()

    def chunk(c, carry):
        b = lax.rem(c, 2)
        bn = lax.rem(c + 1, 2)
        g_desc(c, b).wait()
        d_desc(c, b).wait()

        @pl.when(c >= 1)
        def _():
            s_desc(bn).wait()

        @pl.when(c + 1 < NCHUNK)
        def _():
            g_desc(c + 1, bn).start()
            d_desc(c + 1, bn).start()

        def scale(g, c2):
            nv = norm_v[pl.ds(c * K + 16 * g, 16)]
            for l in range(16):
                e = 16 * g + l
                nb = nv[l]
                for j in range(OUT_FEAT // 16):
                    sl = pl.ds(16 * j, 16)
                    rows_v[b, e, sl] = rows_v[b, e, sl] * nb
            return c2
        lax.fori_loop(0, K // 16, scale, 0)

        pltpu.async_copy(rows_v.at[b], acc.at[dstc_v.at[b]], ssem.at[b],
                         add=True)
        return carry
    lax.fori_loop(0, NCHUNK, chunk, 0)
    s_desc((NCHUNK - 1) % 2).wait()

    plsc.subcore_barrier()

    def wcopy(t, carry):
        j = sid + t * NS
        sl = pl.ds(j * K, K)
        pltpu.sync_copy(acc.at[sl], rows_v.at[0])
        pltpu.sync_copy(rows_v.at[0], part.at[cid, sl])
        return carry
    lax.fori_loop(0, nzc, wcopy, 0)


@jax.jit
def _rgcn_sc(embed, idx2, norm2, dst3):
    mesh = plsc.VectorSubcoreMesh(core_axis_name="c", subcore_axis_name="s")
    return pl.kernel(
        _sc_kernel,
        out_type=jax.ShapeDtypeStruct((NC, N_NODES, OUT_FEAT), jnp.float32),
        mesh=mesh,
        scratch_types=[
            pltpu.VMEM((EPW,), jnp.int32),
            pltpu.VMEM((EPW,), jnp.float32),
            pltpu.VMEM((2, K), jnp.int32),
            pltpu.VMEM((2, K, OUT_FEAT), jnp.float32),
            pltpu.VMEM_SHARED((N_NODES, OUT_FEAT), jnp.float32),
            pltpu.SemaphoreType.DMA((2,)),
            pltpu.SemaphoreType.DMA((2,)),
            pltpu.SemaphoreType.DMA((2,)),
        ],
    )(embed, idx2, norm2, dst3)


def _prep_body(src_ref, rel_ref, o_ref):
    o_ref[...] = rel_ref[...] * IN_FEAT + src_ref[...]


@jax.jit
def _prep(src, rel):
    return pl.pallas_call(
        _prep_body,
        out_shape=jax.ShapeDtypeStruct(src.shape, jnp.int32),
    )(src, rel)


def _add_body(a_ref, b_ref, o_ref):
    o_ref[...] = a_ref[...] + b_ref[...]


@jax.jit
def _combine(part):
    blk = 1000
    spec = pl.BlockSpec((blk, OUT_FEAT), lambda i: (i, 0))
    return pl.pallas_call(
        _add_body,
        out_shape=jax.ShapeDtypeStruct((N_NODES, OUT_FEAT), jnp.float32),
        grid=(N_NODES // blk,),
        in_specs=[spec, spec],
        out_specs=spec,
    )(part[0], part[1])


def kernel(edge_index, rel_type, norm, weight):
    src = edge_index[0].reshape(N_EDGES // OUT_FEAT, OUT_FEAT)
    rel = rel_type.reshape(N_EDGES // OUT_FEAT, OUT_FEAT)
    idx2 = _prep(src, rel).reshape(NW, EPW)
    dst3 = edge_index[1].reshape(NW, NCHUNK, K)
    norm2 = norm.reshape(NW, EPW)
    embed = weight.reshape(NUM_RELS * IN_FEAT, OUT_FEAT)
    part = _rgcn_sc(embed, idx2, norm2, dst3)
    return _combine(part)


# speedup vs baseline: 1.6621x; 1.6621x over previous
import jax
import jax.numpy as jnp
from jax import lax
from jax.experimental import pallas as pl
from jax.experimental.pallas import tpu as pltpu
from jax.experimental.pallas import tpu_sc as plsc

N_NODES = 10000
N_EDGES = 320000
IN_FEAT = 10000
OUT_FEAT = 128
NUM_RELS = 16

NC = 2
NS = 16
NW = NC * NS
EPW = N_EDGES // NW
K = 16
NCHUNK = EPW // K
NBUF = 4
A = 2
ZB = 40
NZCHUNK = N_NODES // ZB


def _sc_kernel(embed, idx2, norm2, dst2, part,
               idx_v, norm_v, dst_v, rows_v, zero_v, acc, gsem, ssem):
    cid = lax.axis_index("c")
    sid = lax.axis_index("s")
    wid = cid * NS + sid

    pltpu.sync_copy(idx2.at[wid], idx_v)
    pltpu.sync_copy(norm2.at[wid], norm_v)
    pltpu.sync_copy(dst2.at[wid], dst_v)

    def zrow(r, carry):
        for j in range(OUT_FEAT // 16):
            zero_v[r, pl.ds(16 * j, 16)] = jnp.zeros((16,), jnp.float32)
        return carry
    lax.fori_loop(0, ZB, zrow, 0)
    nzc = (NZCHUNK - sid + NS - 1) // NS

    def zcopy(t, carry):
        j = sid + t * NS
        pltpu.sync_copy(zero_v, acc.at[pl.ds(j * ZB, ZB)])
        return carry
    lax.fori_loop(0, nzc, zcopy, 0)

    plsc.subcore_barrier()

    def g_desc(c, b):
        ivec = idx_v[pl.ds(c * K, K)]
        return pltpu.make_async_copy(embed.at[ivec], rows_v.at[b],
                                     gsem.at[b])

    def s_desc(c, b):
        dvec = dst_v[pl.ds(c * K, K)]
        return pltpu.make_async_copy(rows_v.at[b], acc.at[dvec],
                                     ssem.at[b])

    def phase(c, b, last):
        rb = rows_v.at[b]
        g_desc(c, b).wait()
        if not last:
            bn = (b + A) % NBUF

            @pl.when(c + A < NCHUNK)
            def _():
                @pl.when(c >= NBUF - A)
                def _():
                    s_desc(c - (NBUF - A), bn).wait()
                g_desc(c + A, bn).start()

        nv = norm_v[pl.ds(c * K, K)]
        for l in range(K):
            nb = nv[l]
            for j in range(OUT_FEAT // 16):
                sl = pl.ds(16 * j, 16)
                rb[l, sl] = rb[l, sl] * nb

        dvec = dst_v[pl.ds(c * K, K)]
        pltpu.async_copy(rb, acc.at[dvec], ssem.at[b], add=True)

    for a in range(A):
        g_desc(a, a % NBUF).start()

    def group(p, carry):
        for b in range(NBUF):
            phase(NBUF * p + b, b, last=False)
        return carry
    lax.fori_loop(0, (NCHUNK - 1) // NBUF, group, 0)

    def tail(c, carry):
        phase(c, (NCHUNK - 1) % NBUF, last=True)
        return carry
    lax.fori_loop(NCHUNK - 1, NCHUNK, tail, 0)

    def drain(c, carry):
        s_desc(c, lax.rem(c, NBUF)).wait()
        return carry
    lax.fori_loop(NCHUNK - (NBUF - A) - A, NCHUNK, drain, 0)

    plsc.subcore_barrier()

    def wcopy(t, carry):
        j = sid + t * NS
        sl = pl.ds(j * ZB, ZB)
        pltpu.sync_copy(acc.at[sl], zero_v)
        pltpu.sync_copy(zero_v, part.at[cid, sl])
        return carry
    lax.fori_loop(0, nzc, wcopy, 0)


@jax.jit
def _rgcn_sc(embed, idx2, norm2, dst2):
    mesh = plsc.VectorSubcoreMesh(core_axis_name="c", subcore_axis_name="s")
    return pl.kernel(
        _sc_kernel,
        out_type=jax.ShapeDtypeStruct((NC, N_NODES, OUT_FEAT), jnp.float32),
        mesh=mesh,
        scratch_types=[
            pltpu.VMEM((EPW,), jnp.int32),
            pltpu.VMEM((EPW,), jnp.float32),
            pltpu.VMEM((EPW,), jnp.int32),
            pltpu.VMEM((NBUF, K, OUT_FEAT), jnp.float32),
            pltpu.VMEM((ZB, OUT_FEAT), jnp.float32),
            pltpu.VMEM_SHARED((N_NODES, OUT_FEAT), jnp.float32),
            pltpu.SemaphoreType.DMA((NBUF,)),
            pltpu.SemaphoreType.DMA((NBUF,)),
        ],
    )(embed, idx2, norm2, dst2)


def _prep_body(src_ref, rel_ref, o_ref):
    o_ref[...] = rel_ref[...] * IN_FEAT + src_ref[...]


@jax.jit
def _prep(src, rel):
    return pl.pallas_call(
        _prep_body,
        out_shape=jax.ShapeDtypeStruct(src.shape, jnp.int32),
    )(src, rel)


def _add_body(a_ref, b_ref, o_ref):
    o_ref[...] = a_ref[...] + b_ref[...]


@jax.jit
def _combine(part):
    blk = 1000
    spec = pl.BlockSpec((blk, OUT_FEAT), lambda i: (i, 0))
    return pl.pallas_call(
        _add_body,
        out_shape=jax.ShapeDtypeStruct((N_NODES, OUT_FEAT), jnp.float32),
        grid=(N_NODES // blk,),
        in_specs=[spec, spec],
        out_specs=spec,
    )(part[0], part[1])


def kernel(edge_index, rel_type, norm, weight):
    src = edge_index[0].reshape(N_EDGES // OUT_FEAT, OUT_FEAT)
    rel = rel_type.reshape(N_EDGES // OUT_FEAT, OUT_FEAT)
    idx2 = _prep(src, rel).reshape(NW, EPW)
    dst2 = edge_index[1].reshape(NW, EPW)
    norm2 = norm.reshape(NW, EPW)
    embed = weight.reshape(NUM_RELS * IN_FEAT, OUT_FEAT)
    part = _rgcn_sc(embed, idx2, norm2, dst2)
    return _combine(part)
